# Initial kernel scaffold; baseline (speedup 1.0000x reference)
import jax
import jax.numpy as jnp
from jax.experimental import pallas as pl


def kernel(x, edge_index, norm, W, V):
    raise NotImplementedError("write your pallas kernel here")


# speedup vs baseline: 12.5034x; 12.5034x over previous
import functools

import jax
import jax.numpy as jnp
from jax import lax
from jax.experimental import pallas as pl
from jax.experimental.pallas import tpu as pltpu
from jax.experimental.pallas import tpu_sc as plsc

N_NODES = 100000
N_EDGES = 1600000
IN_FEA = 128
RANK = 32

NC, NS = 2, 16
NP = 100096
PT = NP // NS
BLK_ROWS = 8
E_PAD = 1605632
IDX_ROWS = E_PAD // 128
ROWS_PER_CORE = IDX_ROWS // NC
ROWS_PER_TILE = ROWS_PER_CORE // NS
N_BLOCKS = ROWS_PER_TILE // BLK_ROWS
A_BLK = 1024
A_GRID = (N_NODES + A_BLK - 1) // A_BLK


def _stage_a(x_ref, w_ref, t0, t1, t2, t3):
    feat = jnp.dot(x_ref[...], w_ref[...], preferred_element_type=jnp.float32)
    mag = jnp.abs(feat)
    logmag = jnp.where(mag > 0.0, jnp.log2(mag), -1e30)
    signbias = jnp.where(feat < 0.0, 3.0, 2.0).astype(jnp.float32)
    t0[...] = logmag[:, :16]
    t1[...] = logmag[:, 16:]
    t2[...] = signbias[:, :16]
    t3[...] = signbias[:, 16:]


def _sc_accumulate(t0, t1, t2, t3, src2d, dst2d, zeros_h, a_out,
                   idx_s, idx_d, rows, acc, gsem, ssem):
    cid = lax.axis_index("c")
    sid = lax.axis_index("s")
    tables = (t0, t1, t2, t3)
    own = pl.ds(sid * PT, PT)
    for r in range(4):
        pltpu.sync_copy(zeros_h.at[own], acc.at[own])
        plsc.subcore_barrier()

        def block_body(b, carry, r=r):
            rb = cid * ROWS_PER_CORE + sid * ROWS_PER_TILE + b * BLK_ROWS
            pltpu.sync_copy(src2d.at[pl.ds(rb, BLK_ROWS)], idx_s)
            pltpu.sync_copy(dst2d.at[pl.ds(rb, BLK_ROWS)], idx_d)
            cps = [
                pltpu.async_copy(tables[r].at[idx_s.at[j]],
                                 rows.at[pl.ds(j * 128, 128)], gsem)
                for j in range(BLK_ROWS)
            ]
            for cp in cps:
                cp.wait()
            cps = [
                pltpu.async_copy(rows.at[pl.ds(j * 128, 128)],
                                 acc.at[idx_d.at[j]], ssem, add=True)
                for j in range(BLK_ROWS)
            ]
            for cp in cps:
                cp.wait()
            return carry

        lax.fori_loop(0, N_BLOCKS, block_body, 0)
        plsc.subcore_barrier()
        base = (r * NC + cid) * NP + sid * PT
        pltpu.sync_copy(acc.at[own], a_out.at[pl.ds(base, PT)])
        plsc.subcore_barrier()


def _stage_c(a_ref, norm_ref, vt_ref, out_ref):
    a = a_ref[...]
    s = a[:, 0, :, :] + a[:, 1, :, :]
    lsum = jnp.concatenate([s[0], s[1]], axis=1)
    bsum = jnp.concatenate([s[2], s[3]], axis=1)
    mag = jnp.exp2(lsum)
    parity = bsum - 2.0 * jnp.floor(bsum * 0.5)
    sign = 1.0 - 2.0 * parity
    neigh = jnp.where(bsum > 0.5, mag * sign, 0.0)
    trans = norm_ref[...][:, None] * neigh
    out_ref[...] = jnp.dot(trans, vt_ref[...],
                           preferred_element_type=jnp.float32)


@jax.jit
def kernel(x, edge_index, norm, W, V):
    t0, t1, t2, t3 = pl.pallas_call(
        _stage_a,
        grid=(A_GRID,),
        in_specs=[
            pl.BlockSpec((A_BLK, IN_FEA), lambda i: (i, 0)),
            pl.BlockSpec((IN_FEA, RANK), lambda i: (0, 0)),
        ],
        out_specs=[pl.BlockSpec((A_BLK, 16), lambda i: (i, 0))] * 4,
        out_shape=[jax.ShapeDtypeStruct((N_NODES, 16), jnp.float32)] * 4,
    )(x, W)

    src = edge_index[0]
    dst = edge_index[1]
    pad = E_PAD - N_EDGES
    src_p = jnp.concatenate([src, jnp.zeros((pad,), jnp.int32)])
    dst_p = jnp.concatenate([dst, jnp.full((pad,), N_NODES, jnp.int32)])
    src2d = src_p.reshape(IDX_ROWS, 128)
    dst2d = dst_p.reshape(IDX_ROWS, 128)
    zeros_h = jnp.zeros((NP, 16), jnp.float32)

    sc = pl.kernel(
        _sc_accumulate,
        out_type=jax.ShapeDtypeStruct((4 * NC * NP, 16), jnp.float32),
        mesh=plsc.VectorSubcoreMesh(core_axis_name="c", subcore_axis_name="s"),
        compiler_params=pltpu.CompilerParams(use_tc_tiling_on_sc=False),
        scratch_types=[
            pltpu.VMEM((BLK_ROWS, 128), jnp.int32),
            pltpu.VMEM((BLK_ROWS, 128), jnp.int32),
            pltpu.VMEM((BLK_ROWS * 128, 16), jnp.float32),
            pltpu.VMEM_SHARED((NP, 16), jnp.float32),
            pltpu.SemaphoreType.DMA,
            pltpu.SemaphoreType.DMA,
        ],
    )
    a_flat = sc(t0, t1, t2, t3, src2d, dst2d, zeros_h)
    a4 = a_flat.reshape(4, NC, NP, 16)

    out = pl.pallas_call(
        _stage_c,
        grid=(A_GRID,),
        in_specs=[
            pl.BlockSpec((4, NC, A_BLK, 16), lambda i: (0, 0, i, 0)),
            pl.BlockSpec((A_BLK,), lambda i: (i,)),
            pl.BlockSpec((RANK, IN_FEA), lambda i: (0, 0)),
        ],
        out_specs=pl.BlockSpec((A_BLK, IN_FEA), lambda i: (i, 0)),
        out_shape=jax.ShapeDtypeStruct((N_NODES, IN_FEA), jnp.float32),
    )(a4, norm.reshape(-1), V.T)
    return out


# speedup vs baseline: 12.5108x; 1.0006x over previous
import functools

import jax
import jax.numpy as jnp
from jax import lax
from jax.experimental import pallas as pl
from jax.experimental.pallas import tpu as pltpu
from jax.experimental.pallas import tpu_sc as plsc

N_NODES = 100000
N_EDGES = 1600000
IN_FEA = 128
RANK = 32

NC, NS = 2, 16
NP = 100096
PT = NP // NS
BLK_ROWS = 8
E_PAD = 1605632
IDX_ROWS = E_PAD // 128
ROWS_PER_CORE = IDX_ROWS // NC
ROWS_PER_TILE = ROWS_PER_CORE // NS
N_BLOCKS = ROWS_PER_TILE // BLK_ROWS
A_BLK = 1024
A_GRID = (N_NODES + A_BLK - 1) // A_BLK


def _stage_a(x_ref, w_ref, t0, t1, t2, t3):
    feat = jnp.dot(x_ref[...], w_ref[...], preferred_element_type=jnp.float32)
    mag = jnp.abs(feat)
    logmag = jnp.where(mag > 0.0, jnp.log2(mag), -1e30)
    signbias = jnp.where(feat < 0.0, 3.0, 2.0).astype(jnp.float32)
    t0[...] = logmag[:, :16]
    t1[...] = logmag[:, 16:]
    t2[...] = signbias[:, :16]
    t3[...] = signbias[:, 16:]


BLK_E = BLK_ROWS * 128


def _sc_accumulate(t0, t1, t2, t3, src1d, dst1d, zeros_h, a_out,
                   idx_s, idx_d, rows, acc, gsem, ssem):
    cid = lax.axis_index("c")
    sid = lax.axis_index("s")
    tables = (t0, t1, t2, t3)
    own = pl.ds(sid * PT, PT)
    tile_rb = cid * ROWS_PER_CORE + sid * ROWS_PER_TILE
    for r in range(4):
        pltpu.sync_copy(zeros_h.at[own], acc.at[own])
        plsc.subcore_barrier()

        def block_body(b, carry, r=r):
            eb = tile_rb * 128 + b * BLK_E
            pltpu.sync_copy(src1d.at[pl.ds(eb, BLK_E)], idx_s)
            pltpu.sync_copy(dst1d.at[pl.ds(eb, BLK_E)], idx_d)
            pltpu.async_copy(tables[r].at[idx_s], rows, gsem).wait()
            pltpu.async_copy(rows, acc.at[idx_d], ssem, add=True).wait()
            return carry

        lax.fori_loop(0, N_BLOCKS, block_body, 0)
        plsc.subcore_barrier()
        base = (r * NC + cid) * NP + sid * PT
        pltpu.sync_copy(acc.at[own], a_out.at[pl.ds(base, PT)])
        plsc.subcore_barrier()


def _stage_c(a_ref, norm_ref, vt_ref, out_ref):
    a = a_ref[...]
    s = a[:, 0, :, :] + a[:, 1, :, :]
    lsum = jnp.concatenate([s[0], s[1]], axis=1)
    bsum = jnp.concatenate([s[2], s[3]], axis=1)
    mag = jnp.exp2(lsum)
    parity = bsum - 2.0 * jnp.floor(bsum * 0.5)
    sign = 1.0 - 2.0 * parity
    neigh = jnp.where(bsum > 0.5, mag * sign, 0.0)
    trans = norm_ref[...][:, None] * neigh
    out_ref[...] = jnp.dot(trans, vt_ref[...],
                           preferred_element_type=jnp.float32)


@jax.jit
def kernel(x, edge_index, norm, W, V):
    t0, t1, t2, t3 = pl.pallas_call(
        _stage_a,
        grid=(A_GRID,),
        in_specs=[
            pl.BlockSpec((A_BLK, IN_FEA), lambda i: (i, 0)),
            pl.BlockSpec((IN_FEA, RANK), lambda i: (0, 0)),
        ],
        out_specs=[pl.BlockSpec((A_BLK, 16), lambda i: (i, 0))] * 4,
        out_shape=[jax.ShapeDtypeStruct((N_NODES, 16), jnp.float32)] * 4,
    )(x, W)

    src = edge_index[0]
    dst = edge_index[1]
    pad = E_PAD - N_EDGES
    src_p = jnp.concatenate([src, jnp.zeros((pad,), jnp.int32)])
    dst_p = jnp.concatenate([dst, jnp.full((pad,), N_NODES, jnp.int32)])
    zeros_h = jnp.zeros((NP, 16), jnp.float32)

    sc = pl.kernel(
        _sc_accumulate,
        out_type=jax.ShapeDtypeStruct((4 * NC * NP, 16), jnp.float32),
        mesh=plsc.VectorSubcoreMesh(core_axis_name="c", subcore_axis_name="s"),
        compiler_params=pltpu.CompilerParams(use_tc_tiling_on_sc=False),
        scratch_types=[
            pltpu.VMEM((BLK_E,), jnp.int32),
            pltpu.VMEM((BLK_E,), jnp.int32),
            pltpu.VMEM((BLK_E, 16), jnp.float32),
            pltpu.VMEM_SHARED((NP, 16), jnp.float32),
            pltpu.SemaphoreType.DMA,
            pltpu.SemaphoreType.DMA,
        ],
    )
    a_flat = sc(t0, t1, t2, t3, src_p, dst_p, zeros_h)
    a4 = a_flat.reshape(4, NC, NP, 16)

    out = pl.pallas_call(
        _stage_c,
        grid=(A_GRID,),
        in_specs=[
            pl.BlockSpec((4, NC, A_BLK, 16), lambda i: (0, 0, i, 0)),
            pl.BlockSpec((A_BLK,), lambda i: (i,)),
            pl.BlockSpec((RANK, IN_FEA), lambda i: (0, 0)),
        ],
        out_specs=pl.BlockSpec((A_BLK, IN_FEA), lambda i: (i, 0)),
        out_shape=jax.ShapeDtypeStruct((N_NODES, IN_FEA), jnp.float32),
    )(a4, norm.reshape(-1), V.T)
    return out
